# baseline (speedup 1.0000x reference)
import functools

import jax
import jax.numpy as jnp
from jax import lax
from jax.experimental import pallas as pl
from jax.experimental.pallas import tpu as pltpu
from jax.experimental.pallas import tpu_sc as plsc

N = 10000
E = 320000
D_IN = 128
D = 64
DC = 16
NCLS = 10

NSC = 1
NTILE = 16
CHUNK = 256
ROWS_T0 = 80
NROWS = NSC * NTILE * ROWS_T0
EPAD = NROWS * CHUNK
NPAD = 10112
NSLICE = NPAD // NTILE

def _make_agg(with_cnt):
    out_type = [jax.ShapeDtypeStruct((NSC, NPAD, D), jnp.float32)]
    if with_cnt:
        out_type.append(jax.ShapeDtypeStruct((NSC, NPAD, DC), jnp.float32))
    scratch = [
        pltpu.VMEM((ROWS_T0, CHUNK), jnp.int32),
        pltpu.VMEM((ROWS_T0, CHUNK), jnp.int32),
        pltpu.VMEM((CHUNK, D), jnp.float32),
        pltpu.VMEM((CHUNK, D), jnp.float32),
        pltpu.VMEM_SHARED((NPAD, D), jnp.float32),
        pltpu.SemaphoreType.DMA,
        pltpu.SemaphoreType.DMA,
        pltpu.SemaphoreType.DMA,
        pltpu.SemaphoreType.DMA,
    ]
    if with_cnt:
        scratch.append(pltpu.VMEM((CHUNK, DC), jnp.float32))
        scratch.append(pltpu.VMEM_SHARED((NPAD, DC), jnp.float32))
        scratch.append(pltpu.SemaphoreType.DMA)
        scratch.append(pltpu.SemaphoreType.DMA)

    def body(*refs):
        if with_cnt:
            (t_hbm, src_hbm, dst_hbm, z64_hbm, z16_hbm, ones_hbm,
             agg_out, cnt_out,
             src_v, dst_v, rows_a, rows_b, acc_sh,
             gsem_a, gsem_b, ssem_a, ssem_b,
             ones_v, cnt_sh, csem_a, csem_b) = refs
        else:
            (t_hbm, src_hbm, dst_hbm, z64_hbm,
             agg_out,
             src_v, dst_v, rows_a, rows_b, acc_sh,
             gsem_a, gsem_b, ssem_a, ssem_b) = refs
        c = lax.axis_index("c")
        s = lax.axis_index("s")
        wid = c * NTILE + s

        zb = s * NSLICE
        pltpu.sync_copy(z64_hbm.at[pl.ds(zb, NSLICE)], acc_sh.at[pl.ds(zb, NSLICE)])
        if with_cnt:
            pltpu.sync_copy(z16_hbm.at[pl.ds(zb, NSLICE)], cnt_sh.at[pl.ds(zb, NSLICE)])
            pltpu.sync_copy(ones_hbm, ones_v)

        my_rows = ROWS_T0
        rb = (c * NTILE + s) * ROWS_T0
        pltpu.sync_copy(src_hbm.at[pl.ds(rb, ROWS_T0)], src_v)
        pltpu.sync_copy(dst_hbm.at[pl.ds(rb, ROWS_T0)], dst_v)
        plsc.subcore_barrier()

        def g_start(k, buf, sem):
            pltpu.async_copy(t_hbm.at[src_v.at[k]], buf, sem)

        def g_wait(buf, sem):
            pltpu.make_async_copy(t_hbm.at[src_v.at[0]], buf, sem).wait()

        def s_start(k, buf, sem):
            pltpu.async_copy(buf, acc_sh.at[dst_v.at[k]], sem, add=True)
            if with_cnt:
                csem = csem_a if sem is ssem_a else csem_b
                pltpu.async_copy(ones_v, cnt_sh.at[dst_v.at[k]], csem, add=True)

        def s_wait(buf, sem):
            pltpu.make_async_copy(buf, acc_sh.at[dst_v.at[0]], sem).wait()
            if with_cnt:
                csem = csem_a if sem is ssem_a else csem_b
                pltpu.make_async_copy(ones_v, cnt_sh.at[dst_v.at[0]], csem).wait()

        g_start(0, rows_a, gsem_a)

        def pipe(i, carry):
            a = 2 * i
            g_wait(rows_a, gsem_a)
            s_start(a, rows_a, ssem_a)

            @pl.when(i > 0)
            def _():
                s_wait(rows_b, ssem_b)

            g_start(a + 1, rows_b, gsem_b)
            g_wait(rows_b, gsem_b)
            s_start(a + 1, rows_b, ssem_b)
            s_wait(rows_a, ssem_a)

            @pl.when(i < my_rows // 2 - 1)
            def _():
                g_start(a + 2, rows_a, gsem_a)

            return carry

        lax.fori_loop(0, my_rows // 2, pipe, 0)
        s_wait(rows_b, ssem_b)
        plsc.subcore_barrier()

        pltpu.sync_copy(acc_sh.at[pl.ds(zb, NSLICE)], agg_out.at[c, pl.ds(zb, NSLICE)])
        if with_cnt:
            pltpu.sync_copy(cnt_sh.at[pl.ds(zb, NSLICE)], cnt_out.at[c, pl.ds(zb, NSLICE)])

    mesh = plsc.VectorSubcoreMesh(core_axis_name="c", subcore_axis_name="s",
                                  num_cores=NSC, num_subcores=NTILE)
    return pl.kernel(
        body, out_type=out_type, mesh=mesh, scratch_types=scratch,
        compiler_params=pltpu.CompilerParams(use_tc_tiling_on_sc=False))


_agg_cnt = functools.cache(lambda: _make_agg(True))
_agg = functools.cache(lambda: _make_agg(False))


_BLK = 1000
_GRID = N // _BLK


def _dot(a, b):
    return jnp.dot(a, b, preferred_element_type=jnp.float32,
                   precision=lax.Precision.HIGHEST)


def _prep_body(x_ref, wl_ref, wr_ref, b_ref, t_ref, r_ref):
    xb = x_ref[...]
    t_ref[...] = _dot(xb, wl_ref[...])
    r_ref[...] = _dot(xb, wr_ref[...]) + b_ref[...]


def _prep(x, wl, wr, b):
    return pl.pallas_call(
        _prep_body,
        grid=(_GRID,),
        in_specs=[
            pl.BlockSpec((_BLK, D_IN), lambda i: (i, 0)),
            pl.BlockSpec((D_IN, D), lambda i: (0, 0)),
            pl.BlockSpec((D_IN, D), lambda i: (0, 0)),
            pl.BlockSpec((1, D), lambda i: (0, 0)),
        ],
        out_specs=[
            pl.BlockSpec((_BLK, D), lambda i: (i, 0)),
            pl.BlockSpec((_BLK, D), lambda i: (i, 0)),
        ],
        out_shape=[
            jax.ShapeDtypeStruct((N, D), jnp.float32),
            jax.ShapeDtypeStruct((N, D), jnp.float32),
        ],
    )(x, wl, wr, b)


def _mean_relu(agg_ref, cnt_ref, r_ref):
    cnt = sum(cnt_ref[i, :, 0:1] for i in range(NSC))
    inv = 1.0 / jnp.maximum(cnt, 1.0)
    mean = sum(agg_ref[i] for i in range(NSC)) * inv
    return jnp.maximum(mean + r_ref[...], 0.0)


def _comb_body(agg_ref, cnt_ref, r_ref, wl_ref, wr_ref, b_ref, t_ref, rn_ref):
    h = _mean_relu(agg_ref, cnt_ref, r_ref)
    t_ref[...] = _dot(h, wl_ref[...])
    rn_ref[...] = _dot(h, wr_ref[...]) + b_ref[...]


def _comb(aggp, cntp, r, wl, wr, b):
    return pl.pallas_call(
        _comb_body,
        grid=(_GRID,),
        in_specs=[
            pl.BlockSpec((NSC, _BLK, D), lambda i: (0, i, 0)),
            pl.BlockSpec((NSC, _BLK, DC), lambda i: (0, i, 0)),
            pl.BlockSpec((_BLK, D), lambda i: (i, 0)),
            pl.BlockSpec((D, D), lambda i: (0, 0)),
            pl.BlockSpec((D, D), lambda i: (0, 0)),
            pl.BlockSpec((1, D), lambda i: (0, 0)),
        ],
        out_specs=[
            pl.BlockSpec((_BLK, D), lambda i: (i, 0)),
            pl.BlockSpec((_BLK, D), lambda i: (i, 0)),
        ],
        out_shape=[
            jax.ShapeDtypeStruct((N, D), jnp.float32),
            jax.ShapeDtypeStruct((N, D), jnp.float32),
        ],
    )(aggp, cntp, r, wl, wr, b)


def _final_body(agg_ref, cnt_ref, r_ref, wc_ref, bc_ref, out_ref):
    h = _mean_relu(agg_ref, cnt_ref, r_ref)
    logits = _dot(h, wc_ref[...]) + bc_ref[...]
    m = jnp.max(logits, axis=1, keepdims=True)
    z = logits - m
    out_ref[...] = z - jnp.log(jnp.sum(jnp.exp(z), axis=1, keepdims=True))


def _final(aggp, cntp, r, wc, bc):
    return pl.pallas_call(
        _final_body,
        grid=(_GRID,),
        in_specs=[
            pl.BlockSpec((NSC, _BLK, D), lambda i: (0, i, 0)),
            pl.BlockSpec((NSC, _BLK, DC), lambda i: (0, i, 0)),
            pl.BlockSpec((_BLK, D), lambda i: (i, 0)),
            pl.BlockSpec((D, NCLS), lambda i: (0, 0)),
            pl.BlockSpec((1, NCLS), lambda i: (0, 0)),
        ],
        out_specs=pl.BlockSpec((_BLK, NCLS), lambda i: (i, 0)),
        out_shape=jax.ShapeDtypeStruct((N, NCLS), jnp.float32),
    )(aggp, cntp, r, wc, bc)


def kernel(x, edge_index, W1l, b1l, W1r, W2l, b2l, W2r, W3l, b3l, W3r, Wc, bc):
    src = edge_index[0].astype(jnp.int32)
    dst = edge_index[1].astype(jnp.int32)
    pad = EPAD - E
    src_p = jnp.concatenate([src, jnp.zeros((pad,), jnp.int32)]).reshape(NROWS, CHUNK)
    dst_p = jnp.concatenate([dst, jnp.full((pad,), N, jnp.int32)]).reshape(NROWS, CHUNK)
    z64 = jnp.zeros((NPAD, D), jnp.float32)
    z16 = jnp.zeros((NPAD, DC), jnp.float32)
    ones = jnp.ones((CHUNK, DC), jnp.float32)
    b1 = b1l.reshape(1, D)
    b2 = b2l.reshape(1, D)
    b3 = b3l.reshape(1, D)
    bcr = bc.reshape(1, NCLS)

    t1, r1 = _prep(x, W1l, W1r, b1)
    aggp1, cntp = _agg_cnt()(t1, src_p, dst_p, z64, z16, ones)
    t2, r2 = _comb(aggp1, cntp, r1, W2l, W2r, b2)
    (aggp2,) = _agg()(t2, src_p, dst_p, z64)
    t3, r3 = _comb(aggp2, cntp, r2, W3l, W3r, b3)
    (aggp3,) = _agg()(t3, src_p, dst_p, z64)
    return _final(aggp3, cntp, r3, Wc, bcr)


# speedup vs baseline: 1.2532x; 1.2532x over previous
import functools

import jax
import jax.numpy as jnp
from jax import lax
from jax.experimental import pallas as pl
from jax.experimental.pallas import tpu as pltpu
from jax.experimental.pallas import tpu_sc as plsc

N = 10000
E = 320000
D_IN = 128
D = 64
DC = 16
NCLS = 10

NSC = 2
NTILE = 16
CHUNK = 256
ROWS_T0 = 58
ROWS_T1 = 22
NROWS = 1344
EPAD = NROWS * CHUNK
NPAD = 10112
NSLICE = NPAD // NTILE

def _make_agg(with_cnt):
    out_type = [jax.ShapeDtypeStruct((NSC, NPAD, D), jnp.float32)]
    if with_cnt:
        out_type.append(jax.ShapeDtypeStruct((NSC, NPAD, DC), jnp.float32))
    scratch = [
        pltpu.VMEM((ROWS_T0, CHUNK), jnp.int32),
        pltpu.VMEM((ROWS_T0, CHUNK), jnp.int32),
        pltpu.VMEM((CHUNK, D), jnp.float32),
        pltpu.VMEM((CHUNK, D), jnp.float32),
        pltpu.VMEM((NSLICE, DC), jnp.float32),
        pltpu.VMEM_SHARED((NPAD, D), jnp.float32),
        pltpu.SemaphoreType.DMA,
        pltpu.SemaphoreType.DMA,
        pltpu.SemaphoreType.DMA,
        pltpu.SemaphoreType.DMA,
    ]
    if with_cnt:
        scratch.append(pltpu.VMEM((CHUNK, DC), jnp.float32))
        scratch.append(pltpu.VMEM_SHARED((NPAD, DC), jnp.float32))
        scratch.append(pltpu.SemaphoreType.DMA)
        scratch.append(pltpu.SemaphoreType.DMA)

    def body(*refs):
        if with_cnt:
            (t_hbm, src_hbm, dst_hbm, z16_hbm, ones_hbm,
             agg_out, cnt_out,
             src_v, dst_v, rows_a, rows_b, zbuf, acc_sh,
             gsem_a, gsem_b, ssem_a, ssem_b,
             ones_v, cnt_sh, csem_a, csem_b) = refs
        else:
            (t_hbm, src_hbm, dst_hbm, z16_hbm,
             agg_out,
             src_v, dst_v, rows_a, rows_b, zbuf, acc_sh,
             gsem_a, gsem_b, ssem_a, ssem_b) = refs
        c = lax.axis_index("c")
        s = lax.axis_index("s")

        zb = s * NSLICE
        pltpu.sync_copy(z16_hbm, zbuf)
        for j in range(D // DC):
            pltpu.sync_copy(
                zbuf, acc_sh.at[pl.ds(zb, NSLICE), pl.ds(j * DC, DC)])
        if with_cnt:
            pltpu.sync_copy(zbuf, cnt_sh.at[pl.ds(zb, NSLICE)])
            pltpu.sync_copy(ones_hbm, ones_v)

        my_rows = jnp.where(c == 0, ROWS_T0, ROWS_T1)
        rb = c * (NTILE * ROWS_T0) + s * my_rows
        pltpu.sync_copy(src_hbm.at[pl.ds(rb, ROWS_T0)], src_v)
        pltpu.sync_copy(dst_hbm.at[pl.ds(rb, ROWS_T0)], dst_v)
        plsc.subcore_barrier()

        def g_start(k, buf, sem):
            pltpu.async_copy(t_hbm.at[src_v.at[k]], buf, sem)

        def g_wait(buf, sem):
            pltpu.make_async_copy(t_hbm.at[src_v.at[0]], buf, sem).wait()

        def s_start(k, buf, sem):
            pltpu.async_copy(buf, acc_sh.at[dst_v.at[k]], sem, add=True)
            if with_cnt:
                csem = csem_a if sem is ssem_a else csem_b
                pltpu.async_copy(ones_v, cnt_sh.at[dst_v.at[k]], csem, add=True)

        def s_wait(buf, sem):
            pltpu.make_async_copy(buf, acc_sh.at[dst_v.at[0]], sem).wait()
            if with_cnt:
                csem = csem_a if sem is ssem_a else csem_b
                pltpu.make_async_copy(ones_v, cnt_sh.at[dst_v.at[0]], csem).wait()

        g_start(0, rows_a, gsem_a)

        def pipe(i, carry):
            a = 2 * i
            g_wait(rows_a, gsem_a)
            s_start(a, rows_a, ssem_a)

            @pl.when(i > 0)
            def _():
                s_wait(rows_b, ssem_b)

            g_start(a + 1, rows_b, gsem_b)
            g_wait(rows_b, gsem_b)
            s_start(a + 1, rows_b, ssem_b)
            s_wait(rows_a, ssem_a)

            @pl.when(i < my_rows // 2 - 1)
            def _():
                g_start(a + 2, rows_a, gsem_a)

            return carry

        lax.fori_loop(0, my_rows // 2, pipe, 0)
        s_wait(rows_b, ssem_b)
        plsc.subcore_barrier()

        pltpu.sync_copy(acc_sh.at[pl.ds(zb, NSLICE)], agg_out.at[c, pl.ds(zb, NSLICE)])
        if with_cnt:
            pltpu.sync_copy(cnt_sh.at[pl.ds(zb, NSLICE)], cnt_out.at[c, pl.ds(zb, NSLICE)])

    mesh = plsc.VectorSubcoreMesh(core_axis_name="c", subcore_axis_name="s",
                                  num_cores=NSC, num_subcores=NTILE)
    return pl.kernel(
        body, out_type=out_type, mesh=mesh, scratch_types=scratch,
        compiler_params=pltpu.CompilerParams(use_tc_tiling_on_sc=False))


_agg_cnt = functools.cache(lambda: _make_agg(True))
_agg = functools.cache(lambda: _make_agg(False))


_BLK = 1000
_GRID = N // _BLK


def _dot(a, b):
    return jnp.dot(a, b, preferred_element_type=jnp.float32,
                   precision=lax.Precision.HIGHEST)


def _prep_body(x_ref, wl_ref, wr_ref, b_ref, t_ref, r_ref):
    xb = x_ref[...]
    t_ref[...] = _dot(xb, wl_ref[...])
    r_ref[...] = _dot(xb, wr_ref[...]) + b_ref[...]


def _prep(x, wl, wr, b):
    return pl.pallas_call(
        _prep_body,
        grid=(_GRID,),
        in_specs=[
            pl.BlockSpec((_BLK, D_IN), lambda i: (i, 0)),
            pl.BlockSpec((D_IN, D), lambda i: (0, 0)),
            pl.BlockSpec((D_IN, D), lambda i: (0, 0)),
            pl.BlockSpec((1, D), lambda i: (0, 0)),
        ],
        out_specs=[
            pl.BlockSpec((_BLK, D), lambda i: (i, 0)),
            pl.BlockSpec((_BLK, D), lambda i: (i, 0)),
        ],
        out_shape=[
            jax.ShapeDtypeStruct((N, D), jnp.float32),
            jax.ShapeDtypeStruct((N, D), jnp.float32),
        ],
    )(x, wl, wr, b)


def _mean_relu(agg_ref, cnt_ref, r_ref):
    cnt = sum(cnt_ref[i, :, 0:1] for i in range(NSC))
    inv = 1.0 / jnp.maximum(cnt, 1.0)
    mean = sum(agg_ref[i] for i in range(NSC)) * inv
    return jnp.maximum(mean + r_ref[...], 0.0)


def _comb_body(agg_ref, cnt_ref, r_ref, wl_ref, wr_ref, b_ref, t_ref, rn_ref):
    h = _mean_relu(agg_ref, cnt_ref, r_ref)
    t_ref[...] = _dot(h, wl_ref[...])
    rn_ref[...] = _dot(h, wr_ref[...]) + b_ref[...]


def _comb(aggp, cntp, r, wl, wr, b):
    return pl.pallas_call(
        _comb_body,
        grid=(_GRID,),
        in_specs=[
            pl.BlockSpec((NSC, _BLK, D), lambda i: (0, i, 0)),
            pl.BlockSpec((NSC, _BLK, DC), lambda i: (0, i, 0)),
            pl.BlockSpec((_BLK, D), lambda i: (i, 0)),
            pl.BlockSpec((D, D), lambda i: (0, 0)),
            pl.BlockSpec((D, D), lambda i: (0, 0)),
            pl.BlockSpec((1, D), lambda i: (0, 0)),
        ],
        out_specs=[
            pl.BlockSpec((_BLK, D), lambda i: (i, 0)),
            pl.BlockSpec((_BLK, D), lambda i: (i, 0)),
        ],
        out_shape=[
            jax.ShapeDtypeStruct((N, D), jnp.float32),
            jax.ShapeDtypeStruct((N, D), jnp.float32),
        ],
    )(aggp, cntp, r, wl, wr, b)


def _final_body(agg_ref, cnt_ref, r_ref, wc_ref, bc_ref, out_ref):
    h = _mean_relu(agg_ref, cnt_ref, r_ref)
    logits = _dot(h, wc_ref[...]) + bc_ref[...]
    m = jnp.max(logits, axis=1, keepdims=True)
    z = logits - m
    out_ref[...] = z - jnp.log(jnp.sum(jnp.exp(z), axis=1, keepdims=True))


def _final(aggp, cntp, r, wc, bc):
    return pl.pallas_call(
        _final_body,
        grid=(_GRID,),
        in_specs=[
            pl.BlockSpec((NSC, _BLK, D), lambda i: (0, i, 0)),
            pl.BlockSpec((NSC, _BLK, DC), lambda i: (0, i, 0)),
            pl.BlockSpec((_BLK, D), lambda i: (i, 0)),
            pl.BlockSpec((D, NCLS), lambda i: (0, 0)),
            pl.BlockSpec((1, NCLS), lambda i: (0, 0)),
        ],
        out_specs=pl.BlockSpec((_BLK, NCLS), lambda i: (i, 0)),
        out_shape=jax.ShapeDtypeStruct((N, NCLS), jnp.float32),
    )(aggp, cntp, r, wc, bc)


def kernel(x, edge_index, W1l, b1l, W1r, W2l, b2l, W2r, W3l, b3l, W3r, Wc, bc):
    src = edge_index[0].astype(jnp.int32)
    dst = edge_index[1].astype(jnp.int32)
    pad = EPAD - E
    src_p = jnp.concatenate([src, jnp.zeros((pad,), jnp.int32)]).reshape(NROWS, CHUNK)
    dst_p = jnp.concatenate([dst, jnp.full((pad,), N, jnp.int32)]).reshape(NROWS, CHUNK)
    z16 = jnp.zeros((NSLICE, DC), jnp.float32)
    ones = jnp.ones((CHUNK, DC), jnp.float32)
    b1 = b1l.reshape(1, D)
    b2 = b2l.reshape(1, D)
    b3 = b3l.reshape(1, D)
    bcr = bc.reshape(1, NCLS)

    t1, r1 = _prep(x, W1l, W1r, b1)
    aggp1, cntp = _agg_cnt()(t1, src_p, dst_p, z16, ones)
    t2, r2 = _comb(aggp1, cntp, r1, W2l, W2r, b2)
    (aggp2,) = _agg()(t2, src_p, dst_p, z16)
    t3, r3 = _comb(aggp2, cntp, r2, W3l, W3r, b3)
    (aggp3,) = _agg()(t3, src_p, dst_p, z16)
    return _final(aggp3, cntp, r3, Wc, bcr)


# speedup vs baseline: 1.4605x; 1.1655x over previous
import functools

import jax
import jax.numpy as jnp
import numpy as np
from jax import lax
from jax.experimental import pallas as pl
from jax.experimental.pallas import tpu as pltpu
from jax.experimental.pallas import tpu_sc as plsc

N = 10000
E = 320000
D_IN = 128
D = 64
DC = 16
NCLS = 10

NSC = 2
NTILE = 16
CHUNK = 256
ROWS_T0 = 58
ROWS_T1 = 22
NROWS = 1344
EPAD = NROWS * CHUNK
NPAD = 10112
NSLICE = NPAD // NTILE

def _make_agg(with_cnt):
    out_type = [jax.ShapeDtypeStruct((NSC, NPAD, D), jnp.float32)]
    if with_cnt:
        out_type.append(jax.ShapeDtypeStruct((NSC, NPAD, DC), jnp.float32))
    scratch = [
        pltpu.VMEM((ROWS_T0, CHUNK), jnp.int32),
        pltpu.VMEM((ROWS_T0, CHUNK), jnp.int32),
        pltpu.VMEM((CHUNK, D), jnp.bfloat16),
        pltpu.VMEM((CHUNK, D), jnp.bfloat16),
        pltpu.VMEM((CHUNK, D), jnp.float32),
        pltpu.VMEM((NSLICE, DC), jnp.float32),
        pltpu.VMEM_SHARED((NPAD, D), jnp.float32),
        pltpu.SemaphoreType.DMA,
        pltpu.SemaphoreType.DMA,
        pltpu.SemaphoreType.DMA,
    ]
    if with_cnt:
        scratch.append(pltpu.VMEM((CHUNK, DC), jnp.float32))
        scratch.append(pltpu.VMEM_SHARED((NPAD, DC), jnp.float32))
        scratch.append(pltpu.SemaphoreType.DMA)

    def body(*refs):
        if with_cnt:
            (t_hbm, src_hbm, dst_hbm, z16_hbm, ones_hbm,
             agg_out, cnt_out,
             src_v, dst_v, bf_a, bf_b, rows_f, zbuf, acc_sh,
             gsem_a, gsem_b, ssem,
             ones_v, cnt_sh, csem) = refs
        else:
            (t_hbm, src_hbm, dst_hbm, z16_hbm,
             agg_out,
             src_v, dst_v, bf_a, bf_b, rows_f, zbuf, acc_sh,
             gsem_a, gsem_b, ssem) = refs
        c = lax.axis_index("c")
        s = lax.axis_index("s")

        zb = s * NSLICE
        pltpu.sync_copy(z16_hbm, zbuf)
        for j in range(D // DC):
            pltpu.sync_copy(
                zbuf, acc_sh.at[pl.ds(zb, NSLICE), pl.ds(j * DC, DC)])
        if with_cnt:
            pltpu.sync_copy(zbuf, cnt_sh.at[pl.ds(zb, NSLICE)])
            pltpu.sync_copy(ones_hbm, ones_v)

        my_rows = jnp.where(c == 0, ROWS_T0, ROWS_T1)
        rb = c * (NTILE * ROWS_T0) + s * my_rows
        pltpu.sync_copy(src_hbm.at[pl.ds(rb, ROWS_T0)], src_v)
        pltpu.sync_copy(dst_hbm.at[pl.ds(rb, ROWS_T0)], dst_v)
        plsc.subcore_barrier()

        def g_start(k, buf, sem):
            pltpu.async_copy(t_hbm.at[src_v.at[k]], buf, sem)

        def g_wait(buf, sem):
            pltpu.make_async_copy(t_hbm.at[src_v.at[0]], buf, sem).wait()

        def s_start(k):
            pltpu.async_copy(rows_f, acc_sh.at[dst_v.at[k]], ssem, add=True)
            if with_cnt:
                pltpu.async_copy(ones_v, cnt_sh.at[dst_v.at[k]], csem, add=True)

        def s_wait():
            pltpu.make_async_copy(rows_f, acc_sh.at[dst_v.at[0]], ssem).wait()
            if with_cnt:
                pltpu.make_async_copy(ones_v, cnt_sh.at[dst_v.at[0]], csem).wait()

        def convert(bfbuf):
            def crow(r, carry):
                for j in range(D // 32):
                    v = bfbuf[r, pl.ds(32 * j, 32)]
                    w = plsc.bitcast(v, jnp.int32)
                    lo = plsc.bitcast(w << 16, jnp.float32)
                    hi = plsc.bitcast(w & jnp.int32(-65536), jnp.float32)
                    rows_f[r, pl.ds(32 * j, 16)] = lo
                    rows_f[r, pl.ds(32 * j + 16, 16)] = hi
                return carry
            lax.fori_loop(0, CHUNK, crow, 0)

        g_start(0, bf_a, gsem_a)
        g_start(1, bf_b, gsem_b)

        def pipe(i, carry):
            a = 2 * i

            g_wait(bf_a, gsem_a)

            @pl.when(i > 0)
            def _():
                s_wait()

            convert(bf_a)

            @pl.when(a + 2 < my_rows)
            def _():
                g_start(a + 2, bf_a, gsem_a)

            s_start(a)

            g_wait(bf_b, gsem_b)
            s_wait()
            convert(bf_b)

            @pl.when(a + 3 < my_rows)
            def _():
                g_start(a + 3, bf_b, gsem_b)

            s_start(a + 1)
            return carry

        lax.fori_loop(0, my_rows // 2, pipe, 0)
        s_wait()
        plsc.subcore_barrier()

        pltpu.sync_copy(acc_sh.at[pl.ds(zb, NSLICE)], agg_out.at[c, pl.ds(zb, NSLICE)])
        if with_cnt:
            pltpu.sync_copy(cnt_sh.at[pl.ds(zb, NSLICE)], cnt_out.at[c, pl.ds(zb, NSLICE)])

    mesh = plsc.VectorSubcoreMesh(core_axis_name="c", subcore_axis_name="s",
                                  num_cores=NSC, num_subcores=NTILE)
    return pl.kernel(
        body, out_type=out_type, mesh=mesh, scratch_types=scratch,
        compiler_params=pltpu.CompilerParams(use_tc_tiling_on_sc=False,
                                             needs_layout_passes=False))


_agg_cnt = functools.cache(lambda: _make_agg(True))
_agg = functools.cache(lambda: _make_agg(False))


_BLK = 1000
_GRID = N // _BLK


def _dot(a, b):
    return jnp.dot(a, b, preferred_element_type=jnp.float32,
                   precision=lax.Precision.HIGHEST)


def _prep_body(x_ref, wl_ref, wr_ref, b_ref, t_ref, r_ref):
    xb = x_ref[...]
    t_ref[...] = _dot(xb, wl_ref[...]).astype(jnp.bfloat16)
    r_ref[...] = _dot(xb, wr_ref[...]) + b_ref[...]


def _prep(x, wl, wr, b):
    return pl.pallas_call(
        _prep_body,
        grid=(_GRID,),
        in_specs=[
            pl.BlockSpec((_BLK, D_IN), lambda i: (i, 0)),
            pl.BlockSpec((D_IN, D), lambda i: (0, 0)),
            pl.BlockSpec((D_IN, D), lambda i: (0, 0)),
            pl.BlockSpec((1, D), lambda i: (0, 0)),
        ],
        out_specs=[
            pl.BlockSpec((_BLK, D), lambda i: (i, 0)),
            pl.BlockSpec((_BLK, D), lambda i: (i, 0)),
        ],
        out_shape=[
            jax.ShapeDtypeStruct((N, D), jnp.bfloat16),
            jax.ShapeDtypeStruct((N, D), jnp.float32),
        ],
    )(x, wl, wr, b)


def _mean_relu(agg_ref, cnt_ref, r_ref):
    cnt = sum(cnt_ref[i, :, 0:1] for i in range(NSC))
    inv = 1.0 / jnp.maximum(cnt, 1.0)
    mean = sum(agg_ref[i] for i in range(NSC)) * inv
    return jnp.maximum(mean + r_ref[...], 0.0)


def _comb_body(agg_ref, cnt_ref, r_ref, wl_ref, wr_ref, b_ref, t_ref, rn_ref):
    h = _mean_relu(agg_ref, cnt_ref, r_ref)
    t_ref[...] = _dot(h, wl_ref[...]).astype(jnp.bfloat16)
    rn_ref[...] = _dot(h, wr_ref[...]) + b_ref[...]


def _comb(aggp, cntp, r, wl, wr, b):
    return pl.pallas_call(
        _comb_body,
        grid=(_GRID,),
        in_specs=[
            pl.BlockSpec((NSC, _BLK, D), lambda i: (0, i, 0)),
            pl.BlockSpec((NSC, _BLK, DC), lambda i: (0, i, 0)),
            pl.BlockSpec((_BLK, D), lambda i: (i, 0)),
            pl.BlockSpec((D, D), lambda i: (0, 0)),
            pl.BlockSpec((D, D), lambda i: (0, 0)),
            pl.BlockSpec((1, D), lambda i: (0, 0)),
        ],
        out_specs=[
            pl.BlockSpec((_BLK, D), lambda i: (i, 0)),
            pl.BlockSpec((_BLK, D), lambda i: (i, 0)),
        ],
        out_shape=[
            jax.ShapeDtypeStruct((N, D), jnp.bfloat16),
            jax.ShapeDtypeStruct((N, D), jnp.float32),
        ],
    )(aggp, cntp, r, wl, wr, b)


def _final_body(agg_ref, cnt_ref, r_ref, wc_ref, bc_ref, out_ref):
    h = _mean_relu(agg_ref, cnt_ref, r_ref)
    logits = _dot(h, wc_ref[...]) + bc_ref[...]
    m = jnp.max(logits, axis=1, keepdims=True)
    z = logits - m
    out_ref[...] = z - jnp.log(jnp.sum(jnp.exp(z), axis=1, keepdims=True))


def _final(aggp, cntp, r, wc, bc):
    return pl.pallas_call(
        _final_body,
        grid=(_GRID,),
        in_specs=[
            pl.BlockSpec((NSC, _BLK, D), lambda i: (0, i, 0)),
            pl.BlockSpec((NSC, _BLK, DC), lambda i: (0, i, 0)),
            pl.BlockSpec((_BLK, D), lambda i: (i, 0)),
            pl.BlockSpec((D, NCLS), lambda i: (0, 0)),
            pl.BlockSpec((1, NCLS), lambda i: (0, 0)),
        ],
        out_specs=pl.BlockSpec((_BLK, NCLS), lambda i: (i, 0)),
        out_shape=jax.ShapeDtypeStruct((N, NCLS), jnp.float32),
    )(aggp, cntp, r, wc, bc)


_PERM = np.empty((D,), np.int32)
for _j in (0, 32):
    for _m in range(16):
        _PERM[_j + 2 * _m] = _j + _m
        _PERM[_j + 2 * _m + 1] = _j + 16 + _m
_PERM_J = tuple(int(v) for v in _PERM)


def kernel(x, edge_index, W1l, b1l, W1r, W2l, b2l, W2r, W3l, b3l, W3r, Wc, bc):
    src = edge_index[0].astype(jnp.int32)
    dst = edge_index[1].astype(jnp.int32)
    perm = jnp.asarray(_PERM_J, dtype=jnp.int32)
    W1l = W1l[:, perm]
    W2l = W2l[:, perm]
    W3l = W3l[:, perm]
    pad = EPAD - E
    src_p = jnp.concatenate([src, jnp.zeros((pad,), jnp.int32)]).reshape(NROWS, CHUNK)
    dst_p = jnp.concatenate([dst, jnp.full((pad,), N, jnp.int32)]).reshape(NROWS, CHUNK)
    z16 = jnp.zeros((NSLICE, DC), jnp.float32)
    ones = jnp.ones((CHUNK, DC), jnp.float32)
    b1 = b1l.reshape(1, D)
    b2 = b2l.reshape(1, D)
    b3 = b3l.reshape(1, D)
    bcr = bc.reshape(1, NCLS)

    t1, r1 = _prep(x, W1l, W1r, b1)
    aggp1, cntp = _agg_cnt()(t1, src_p, dst_p, z16, ones)
    t2, r2 = _comb(aggp1, cntp, r1, W2l, W2r, b2)
    (aggp2,) = _agg()(t2, src_p, dst_p, z16)
    t3, r3 = _comb(aggp2, cntp, r2, W3l, W3r, b3)
    (aggp3,) = _agg()(t3, src_p, dst_p, z16)
    return _final(aggp3, cntp, r3, Wc, bcr)


# speedup vs baseline: 1.6119x; 1.1036x over previous
import functools

import jax
import jax.numpy as jnp
import numpy as np
from jax import lax
from jax.experimental import pallas as pl
from jax.experimental.pallas import tpu as pltpu
from jax.experimental.pallas import tpu_sc as plsc

N = 10000
E = 320000
D_IN = 128
D = 64
DC = 16
NCLS = 10

NSC = 2
NTILE = 16
CHUNK = 256
ROWS_T0 = 52
ROWS_T1 = 28
NROWS = 1344
EPAD = NROWS * CHUNK
NPAD = 10112
NSLICE = NPAD // NTILE

def _make_agg(with_cnt):
    out_type = [jax.ShapeDtypeStruct((NSC, NPAD, D), jnp.float32)]
    if with_cnt:
        out_type.append(jax.ShapeDtypeStruct((NSC, NPAD, DC), jnp.float32))
    scratch = [
        pltpu.VMEM((ROWS_T0, CHUNK), jnp.int32),
        pltpu.VMEM((ROWS_T0, CHUNK), jnp.int32),
        pltpu.VMEM((CHUNK, D), jnp.bfloat16),
        pltpu.VMEM((CHUNK, D), jnp.bfloat16),
        pltpu.VMEM((CHUNK, D), jnp.float32),
        pltpu.VMEM((CHUNK, D), jnp.float32),
        pltpu.VMEM_SHARED((NPAD, D), jnp.float32),
        pltpu.SemaphoreType.DMA,
        pltpu.SemaphoreType.DMA,
        pltpu.SemaphoreType.DMA,
        pltpu.SemaphoreType.DMA,
    ]
    if with_cnt:
        scratch.append(pltpu.VMEM((CHUNK, DC), jnp.float32))
        scratch.append(pltpu.VMEM_SHARED((NPAD, DC), jnp.float32))
        scratch.append(pltpu.SemaphoreType.DMA)
        scratch.append(pltpu.SemaphoreType.DMA)

    def body(*refs):
        if with_cnt:
            (t_hbm, src_hbm, dst_hbm, z16_hbm, ones_hbm,
             agg_out, cnt_out,
             src_v, dst_v, bf_a, bf_b, f0, f1, acc_sh,
             gsem_a, gsem_b, ssem0, ssem1,
             ones_v, cnt_sh, csem0, csem1) = refs
        else:
            (t_hbm, src_hbm, dst_hbm, z16_hbm,
             agg_out,
             src_v, dst_v, bf_a, bf_b, f0, f1, acc_sh,
             gsem_a, gsem_b, ssem0, ssem1) = refs
            csem0 = csem1 = None
        c = lax.axis_index("c")
        s = lax.axis_index("s")

        zb = s * NSLICE
        for j in range(D // DC):
            pltpu.sync_copy(
                z16_hbm, acc_sh.at[pl.ds(zb, NSLICE), pl.ds(j * DC, DC)])
        if with_cnt:
            pltpu.sync_copy(z16_hbm, cnt_sh.at[pl.ds(zb, NSLICE)])
            pltpu.sync_copy(ones_hbm, ones_v)

        my_rows = jnp.where(c == 0, ROWS_T0, ROWS_T1)
        rb = c * (NTILE * ROWS_T0) + s * my_rows
        pltpu.sync_copy(src_hbm.at[pl.ds(rb, ROWS_T0)], src_v)
        pltpu.sync_copy(dst_hbm.at[pl.ds(rb, ROWS_T0)], dst_v)
        plsc.subcore_barrier()

        def g_start(k, buf, sem):
            pltpu.async_copy(t_hbm.at[src_v.at[k]], buf, sem)

        def g_wait(buf, sem):
            pltpu.make_async_copy(t_hbm.at[src_v.at[0]], buf, sem).wait()

        def s_start(k, fbuf, sem, cs):
            pltpu.async_copy(fbuf, acc_sh.at[dst_v.at[k]], sem, add=True)
            if with_cnt:
                pltpu.async_copy(ones_v, cnt_sh.at[dst_v.at[k]], cs, add=True)

        def s_wait(fbuf, sem, cs):
            pltpu.make_async_copy(fbuf, acc_sh.at[dst_v.at[0]], sem).wait()
            if with_cnt:
                pltpu.make_async_copy(ones_v, cnt_sh.at[dst_v.at[0]], cs).wait()

        def convert(bfbuf, fbuf):
            def crow(q, carry):
                for u in range(4):
                    r = 4 * q + u
                    for j in range(D // 32):
                        v = bfbuf[r, pl.ds(32 * j, 32)]
                        w = plsc.bitcast(v, jnp.int32)
                        lo = plsc.bitcast(w << 16, jnp.float32)
                        hi = plsc.bitcast(w & jnp.int32(-65536), jnp.float32)
                        fbuf[r, pl.ds(32 * j, 16)] = lo
                        fbuf[r, pl.ds(32 * j + 16, 16)] = hi
                return carry
            lax.fori_loop(0, CHUNK // 4, crow, 0)

        g_start(0, bf_a, gsem_a)
        g_start(1, bf_b, gsem_b)

        def pipe(i, carry):
            a = 2 * i

            g_wait(bf_a, gsem_a)

            @pl.when(i > 0)
            def _():
                s_wait(f0, ssem0, csem0)

            convert(bf_a, f0)

            @pl.when(a + 2 < my_rows)
            def _():
                g_start(a + 2, bf_a, gsem_a)

            s_start(a, f0, ssem0, csem0)

            g_wait(bf_b, gsem_b)

            @pl.when(i > 0)
            def _():
                s_wait(f1, ssem1, csem1)

            convert(bf_b, f1)

            @pl.when(a + 3 < my_rows)
            def _():
                g_start(a + 3, bf_b, gsem_b)

            s_start(a + 1, f1, ssem1, csem1)
            return carry

        lax.fori_loop(0, my_rows // 2, pipe, 0)
        s_wait(f0, ssem0, csem0)
        s_wait(f1, ssem1, csem1)
        plsc.subcore_barrier()

        pltpu.sync_copy(acc_sh.at[pl.ds(zb, NSLICE)], agg_out.at[c, pl.ds(zb, NSLICE)])
        if with_cnt:
            pltpu.sync_copy(cnt_sh.at[pl.ds(zb, NSLICE)], cnt_out.at[c, pl.ds(zb, NSLICE)])

    mesh = plsc.VectorSubcoreMesh(core_axis_name="c", subcore_axis_name="s",
                                  num_cores=NSC, num_subcores=NTILE)
    return pl.kernel(
        body, out_type=out_type, mesh=mesh, scratch_types=scratch,
        compiler_params=pltpu.CompilerParams(use_tc_tiling_on_sc=False,
                                             needs_layout_passes=False))


_agg_cnt = functools.cache(lambda: _make_agg(True))
_agg = functools.cache(lambda: _make_agg(False))


_BLK = 1000
_GRID = N // _BLK


def _dot(a, b):
    return jnp.dot(a, b, preferred_element_type=jnp.float32,
                   precision=lax.Precision.HIGHEST)


def _prep_body(x_ref, wl_ref, wr_ref, b_ref, t_ref, r_ref):
    xb = x_ref[...]
    t_ref[...] = _dot(xb, wl_ref[...]).astype(jnp.bfloat16)
    r_ref[...] = _dot(xb, wr_ref[...]) + b_ref[...]


def _prep(x, wl, wr, b):
    return pl.pallas_call(
        _prep_body,
        grid=(_GRID,),
        in_specs=[
            pl.BlockSpec((_BLK, D_IN), lambda i: (i, 0)),
            pl.BlockSpec((D_IN, D), lambda i: (0, 0)),
            pl.BlockSpec((D_IN, D), lambda i: (0, 0)),
            pl.BlockSpec((1, D), lambda i: (0, 0)),
        ],
        out_specs=[
            pl.BlockSpec((_BLK, D), lambda i: (i, 0)),
            pl.BlockSpec((_BLK, D), lambda i: (i, 0)),
        ],
        out_shape=[
            jax.ShapeDtypeStruct((N, D), jnp.bfloat16),
            jax.ShapeDtypeStruct((N, D), jnp.float32),
        ],
    )(x, wl, wr, b)


def _mean_relu(agg_ref, cnt_ref, r_ref):
    cnt = sum(cnt_ref[i, :, 0:1] for i in range(NSC))
    inv = 1.0 / jnp.maximum(cnt, 1.0)
    mean = sum(agg_ref[i] for i in range(NSC)) * inv
    return jnp.maximum(mean + r_ref[...], 0.0)


def _comb_body(agg_ref, cnt_ref, r_ref, wl_ref, wr_ref, b_ref, t_ref, rn_ref):
    h = _mean_relu(agg_ref, cnt_ref, r_ref)
    t_ref[...] = _dot(h, wl_ref[...]).astype(jnp.bfloat16)
    rn_ref[...] = _dot(h, wr_ref[...]) + b_ref[...]


def _comb(aggp, cntp, r, wl, wr, b):
    return pl.pallas_call(
        _comb_body,
        grid=(_GRID,),
        in_specs=[
            pl.BlockSpec((NSC, _BLK, D), lambda i: (0, i, 0)),
            pl.BlockSpec((NSC, _BLK, DC), lambda i: (0, i, 0)),
            pl.BlockSpec((_BLK, D), lambda i: (i, 0)),
            pl.BlockSpec((D, D), lambda i: (0, 0)),
            pl.BlockSpec((D, D), lambda i: (0, 0)),
            pl.BlockSpec((1, D), lambda i: (0, 0)),
        ],
        out_specs=[
            pl.BlockSpec((_BLK, D), lambda i: (i, 0)),
            pl.BlockSpec((_BLK, D), lambda i: (i, 0)),
        ],
        out_shape=[
            jax.ShapeDtypeStruct((N, D), jnp.bfloat16),
            jax.ShapeDtypeStruct((N, D), jnp.float32),
        ],
    )(aggp, cntp, r, wl, wr, b)


def _final_body(agg_ref, cnt_ref, r_ref, wc_ref, bc_ref, out_ref):
    h = _mean_relu(agg_ref, cnt_ref, r_ref)
    logits = _dot(h, wc_ref[...]) + bc_ref[...]
    m = jnp.max(logits, axis=1, keepdims=True)
    z = logits - m
    out_ref[...] = z - jnp.log(jnp.sum(jnp.exp(z), axis=1, keepdims=True))


def _final(aggp, cntp, r, wc, bc):
    return pl.pallas_call(
        _final_body,
        grid=(_GRID,),
        in_specs=[
            pl.BlockSpec((NSC, _BLK, D), lambda i: (0, i, 0)),
            pl.BlockSpec((NSC, _BLK, DC), lambda i: (0, i, 0)),
            pl.BlockSpec((_BLK, D), lambda i: (i, 0)),
            pl.BlockSpec((D, NCLS), lambda i: (0, 0)),
            pl.BlockSpec((1, NCLS), lambda i: (0, 0)),
        ],
        out_specs=pl.BlockSpec((_BLK, NCLS), lambda i: (i, 0)),
        out_shape=jax.ShapeDtypeStruct((N, NCLS), jnp.float32),
    )(aggp, cntp, r, wc, bc)


_PERM = np.empty((D,), np.int32)
for _j in (0, 32):
    for _m in range(16):
        _PERM[_j + 2 * _m] = _j + _m
        _PERM[_j + 2 * _m + 1] = _j + 16 + _m
_PERM_J = tuple(int(v) for v in _PERM)


def kernel(x, edge_index, W1l, b1l, W1r, W2l, b2l, W2r, W3l, b3l, W3r, Wc, bc):
    src = edge_index[0].astype(jnp.int32)
    dst = edge_index[1].astype(jnp.int32)
    perm = jnp.asarray(_PERM_J, dtype=jnp.int32)
    W1l = W1l[:, perm]
    W2l = W2l[:, perm]
    W3l = W3l[:, perm]
    pad = EPAD - E
    src_p = jnp.concatenate([src, jnp.zeros((pad,), jnp.int32)]).reshape(NROWS, CHUNK)
    dst_p = jnp.concatenate([dst, jnp.full((pad,), N, jnp.int32)]).reshape(NROWS, CHUNK)
    z16 = jnp.zeros((NSLICE, DC), jnp.float32)
    ones = jnp.ones((CHUNK, DC), jnp.float32)
    b1 = b1l.reshape(1, D)
    b2 = b2l.reshape(1, D)
    b3 = b3l.reshape(1, D)
    bcr = bc.reshape(1, NCLS)

    t1, r1 = _prep(x, W1l, W1r, b1)
    aggp1, cntp = _agg_cnt()(t1, src_p, dst_p, z16, ones)
    t2, r2 = _comb(aggp1, cntp, r1, W2l, W2r, b2)
    (aggp2,) = _agg()(t2, src_p, dst_p, z16)
    t3, r3 = _comb(aggp2, cntp, r2, W3l, W3r, b3)
    (aggp3,) = _agg()(t3, src_p, dst_p, z16)
    return _final(aggp3, cntp, r3, Wc, bcr)


# speedup vs baseline: 1.7692x; 1.0976x over previous
import functools

import jax
import jax.numpy as jnp
import numpy as np
from jax import lax
from jax.experimental import pallas as pl
from jax.experimental.pallas import tpu as pltpu
from jax.experimental.pallas import tpu_sc as plsc

N = 10000
E = 320000
D_IN = 128
D = 64
DC = 16
NCLS = 10

NSC = 2
NTILE = 16
CHUNK = 256
ROWS_T0 = 48
ROWS_T1 = 32
NROWS = 1344
EPAD = NROWS * CHUNK
NPAD = 10112
NSLICE = NPAD // NTILE

def _make_agg(with_cnt):
    out_type = [jax.ShapeDtypeStruct((NSC, NPAD, D), jnp.float32)]
    if with_cnt:
        out_type.append(jax.ShapeDtypeStruct((NSC, NPAD, DC), jnp.float32))
    scratch = [
        pltpu.VMEM((ROWS_T0, CHUNK), jnp.int32),
        pltpu.VMEM((ROWS_T0, CHUNK), jnp.int32),
        pltpu.VMEM((CHUNK, D), jnp.bfloat16),
        pltpu.VMEM((CHUNK, D), jnp.bfloat16),
        pltpu.VMEM((CHUNK, D), jnp.float32),
        pltpu.VMEM((CHUNK, D), jnp.float32),
        pltpu.VMEM_SHARED((NPAD, D), jnp.float32),
        pltpu.SemaphoreType.DMA,
        pltpu.SemaphoreType.DMA,
        pltpu.SemaphoreType.DMA,
        pltpu.SemaphoreType.DMA,
    ]
    if with_cnt:
        scratch.append(pltpu.VMEM((CHUNK, DC), jnp.float32))
        scratch.append(pltpu.VMEM_SHARED((NPAD, DC), jnp.float32))
        scratch.append(pltpu.SemaphoreType.DMA)
        scratch.append(pltpu.SemaphoreType.DMA)

    def body(*refs):
        if with_cnt:
            (t_hbm, src_hbm, dst_hbm, z16_hbm, ones_hbm,
             agg_out, cnt_out,
             src_v, dst_v, bf_a, bf_b, f0, f1, acc_sh,
             gsem_a, gsem_b, ssem0, ssem1,
             ones_v, cnt_sh, csem0, csem1) = refs
        else:
            (t_hbm, src_hbm, dst_hbm, z16_hbm,
             agg_out,
             src_v, dst_v, bf_a, bf_b, f0, f1, acc_sh,
             gsem_a, gsem_b, ssem0, ssem1) = refs
            csem0 = csem1 = None
        c = lax.axis_index("c")
        s = lax.axis_index("s")

        zb = s * NSLICE
        for j in range(D // DC):
            pltpu.sync_copy(
                z16_hbm, acc_sh.at[pl.ds(zb, NSLICE), pl.ds(j * DC, DC)])
        if with_cnt:
            pltpu.sync_copy(z16_hbm, cnt_sh.at[pl.ds(zb, NSLICE)])
            pltpu.sync_copy(ones_hbm, ones_v)

        my_rows = jnp.where(c == 0, ROWS_T0, ROWS_T1)
        rb = c * (NTILE * ROWS_T0) + s * my_rows
        pltpu.sync_copy(src_hbm.at[pl.ds(rb, ROWS_T0)], src_v)
        pltpu.sync_copy(dst_hbm.at[pl.ds(rb, ROWS_T0)], dst_v)
        plsc.subcore_barrier()

        def g_start(k, buf, sem):
            pltpu.async_copy(t_hbm.at[src_v.at[k]], buf, sem)

        def g_wait(buf, sem):
            pltpu.make_async_copy(t_hbm.at[src_v.at[0]], buf, sem).wait()

        def s_start(k, fbuf, sem, cs):
            pltpu.async_copy(fbuf, acc_sh.at[dst_v.at[k]], sem, add=True)
            if with_cnt:
                pltpu.async_copy(ones_v, cnt_sh.at[dst_v.at[k]], cs, add=True)

        def s_wait(fbuf, sem, cs):
            pltpu.make_async_copy(fbuf, acc_sh.at[dst_v.at[0]], sem).wait()
            if with_cnt:
                pltpu.make_async_copy(ones_v, cnt_sh.at[dst_v.at[0]], cs).wait()

        def convert(bfbuf, fbuf):
            def crow(q, carry):
                for u in range(4):
                    r = 4 * q + u
                    for j in range(D // 32):
                        v = bfbuf[r, pl.ds(32 * j, 32)]
                        w = plsc.bitcast(v, jnp.int32)
                        lo = plsc.bitcast(w << 16, jnp.float32)
                        hi = plsc.bitcast(w & jnp.int32(-65536), jnp.float32)
                        fbuf[r, pl.ds(32 * j, 16)] = lo
                        fbuf[r, pl.ds(32 * j + 16, 16)] = hi
                return carry
            lax.fori_loop(0, CHUNK // 4, crow, 0)

        g_start(0, bf_a, gsem_a)
        g_start(1, bf_b, gsem_b)

        def pipe(i, carry):
            a = 2 * i

            g_wait(bf_a, gsem_a)

            @pl.when(i > 0)
            def _():
                s_wait(f0, ssem0, csem0)

            convert(bf_a, f0)

            @pl.when(a + 2 < my_rows)
            def _():
                g_start(a + 2, bf_a, gsem_a)

            s_start(a, f0, ssem0, csem0)

            g_wait(bf_b, gsem_b)

            @pl.when(i > 0)
            def _():
                s_wait(f1, ssem1, csem1)

            convert(bf_b, f1)

            @pl.when(a + 3 < my_rows)
            def _():
                g_start(a + 3, bf_b, gsem_b)

            s_start(a + 1, f1, ssem1, csem1)
            return carry

        lax.fori_loop(0, my_rows // 2, pipe, 0)
        s_wait(f0, ssem0, csem0)
        s_wait(f1, ssem1, csem1)
        plsc.subcore_barrier()

        pltpu.sync_copy(acc_sh.at[pl.ds(zb, NSLICE)], agg_out.at[c, pl.ds(zb, NSLICE)])
        if with_cnt:
            pltpu.sync_copy(cnt_sh.at[pl.ds(zb, NSLICE)], cnt_out.at[c, pl.ds(zb, NSLICE)])

    mesh = plsc.VectorSubcoreMesh(core_axis_name="c", subcore_axis_name="s",
                                  num_cores=NSC, num_subcores=NTILE)
    return pl.kernel(
        body, out_type=out_type, mesh=mesh, scratch_types=scratch,
        compiler_params=pltpu.CompilerParams(use_tc_tiling_on_sc=False,
                                             needs_layout_passes=False))


_agg_cnt = functools.cache(lambda: _make_agg(True))
_agg = functools.cache(lambda: _make_agg(False))


_BLK = 2000
_GRID = N // _BLK


def _dot(a, b):
    return jnp.dot(a, b, preferred_element_type=jnp.float32,
                   precision=lax.Precision.HIGHEST)


def _prep_body(x_ref, wl_ref, wr_ref, b_ref, t_ref, r_ref):
    xb = x_ref[...]
    t_ref[...] = _dot(xb, wl_ref[...]).astype(jnp.bfloat16)
    r_ref[...] = _dot(xb, wr_ref[...]) + b_ref[...]


def _prep(x, wl, wr, b):
    return pl.pallas_call(
        _prep_body,
        grid=(_GRID,),
        in_specs=[
            pl.BlockSpec((_BLK, D_IN), lambda i: (i, 0)),
            pl.BlockSpec((D_IN, D), lambda i: (0, 0)),
            pl.BlockSpec((D_IN, D), lambda i: (0, 0)),
            pl.BlockSpec((1, D), lambda i: (0, 0)),
        ],
        out_specs=[
            pl.BlockSpec((_BLK, D), lambda i: (i, 0)),
            pl.BlockSpec((_BLK, D), lambda i: (i, 0)),
        ],
        out_shape=[
            jax.ShapeDtypeStruct((N, D), jnp.bfloat16),
            jax.ShapeDtypeStruct((N, D), jnp.float32),
        ],
    )(x, wl, wr, b)


def _mean_relu(agg_ref, cnt_ref, r_ref):
    cnt = sum(cnt_ref[i, :, 0:1] for i in range(NSC))
    inv = 1.0 / jnp.maximum(cnt, 1.0)
    mean = sum(agg_ref[i] for i in range(NSC)) * inv
    return jnp.maximum(mean + r_ref[...], 0.0)


def _comb_body(agg_ref, cnt_ref, r_ref, wl_ref, wr_ref, b_ref, t_ref, rn_ref):
    h = _mean_relu(agg_ref, cnt_ref, r_ref)
    t_ref[...] = _dot(h, wl_ref[...]).astype(jnp.bfloat16)
    rn_ref[...] = _dot(h, wr_ref[...]) + b_ref[...]


def _comb(aggp, cntp, r, wl, wr, b):
    return pl.pallas_call(
        _comb_body,
        grid=(_GRID,),
        in_specs=[
            pl.BlockSpec((NSC, _BLK, D), lambda i: (0, i, 0)),
            pl.BlockSpec((NSC, _BLK, DC), lambda i: (0, i, 0)),
            pl.BlockSpec((_BLK, D), lambda i: (i, 0)),
            pl.BlockSpec((D, D), lambda i: (0, 0)),
            pl.BlockSpec((D, D), lambda i: (0, 0)),
            pl.BlockSpec((1, D), lambda i: (0, 0)),
        ],
        out_specs=[
            pl.BlockSpec((_BLK, D), lambda i: (i, 0)),
            pl.BlockSpec((_BLK, D), lambda i: (i, 0)),
        ],
        out_shape=[
            jax.ShapeDtypeStruct((N, D), jnp.bfloat16),
            jax.ShapeDtypeStruct((N, D), jnp.float32),
        ],
    )(aggp, cntp, r, wl, wr, b)


def _final_body(agg_ref, cnt_ref, r_ref, wc_ref, bc_ref, out_ref):
    h = _mean_relu(agg_ref, cnt_ref, r_ref)
    logits = _dot(h, wc_ref[...]) + bc_ref[...]
    m = jnp.max(logits, axis=1, keepdims=True)
    z = logits - m
    out_ref[...] = z - jnp.log(jnp.sum(jnp.exp(z), axis=1, keepdims=True))


def _final(aggp, cntp, r, wc, bc):
    return pl.pallas_call(
        _final_body,
        grid=(_GRID,),
        in_specs=[
            pl.BlockSpec((NSC, _BLK, D), lambda i: (0, i, 0)),
            pl.BlockSpec((NSC, _BLK, DC), lambda i: (0, i, 0)),
            pl.BlockSpec((_BLK, D), lambda i: (i, 0)),
            pl.BlockSpec((D, NCLS), lambda i: (0, 0)),
            pl.BlockSpec((1, NCLS), lambda i: (0, 0)),
        ],
        out_specs=pl.BlockSpec((_BLK, NCLS), lambda i: (i, 0)),
        out_shape=jax.ShapeDtypeStruct((N, NCLS), jnp.float32),
    )(aggp, cntp, r, wc, bc)


_PERM = np.empty((D,), np.int32)
for _j in (0, 32):
    for _m in range(16):
        _PERM[_j + 2 * _m] = _j + _m
        _PERM[_j + 2 * _m + 1] = _j + 16 + _m
_PERM_J = tuple(int(v) for v in _PERM)


def kernel(x, edge_index, W1l, b1l, W1r, W2l, b2l, W2r, W3l, b3l, W3r, Wc, bc):
    src = edge_index[0].astype(jnp.int32)
    dst = edge_index[1].astype(jnp.int32)
    perm = jnp.asarray(_PERM_J, dtype=jnp.int32)
    W1l = W1l[:, perm]
    W2l = W2l[:, perm]
    W3l = W3l[:, perm]
    pad = EPAD - E
    src_p = jnp.concatenate([src, jnp.zeros((pad,), jnp.int32)]).reshape(NROWS, CHUNK)
    dst_p = jnp.concatenate([dst, jnp.full((pad,), N, jnp.int32)]).reshape(NROWS, CHUNK)
    z16 = jnp.zeros((NSLICE, DC), jnp.float32)
    ones = jnp.ones((CHUNK, DC), jnp.float32)
    b1 = b1l.reshape(1, D)
    b2 = b2l.reshape(1, D)
    b3 = b3l.reshape(1, D)
    bcr = bc.reshape(1, NCLS)

    t1, r1 = _prep(x, W1l, W1r, b1)
    aggp1, cntp = _agg_cnt()(t1, src_p, dst_p, z16, ones)
    t2, r2 = _comb(aggp1, cntp, r1, W2l, W2r, b2)
    (aggp2,) = _agg()(t2, src_p, dst_p, z16)
    t3, r3 = _comb(aggp2, cntp, r2, W3l, W3r, b3)
    (aggp3,) = _agg()(t3, src_p, dst_p, z16)
    return _final(aggp3, cntp, r3, Wc, bcr)
